# baseline (speedup 1.0000x reference)
import functools

import jax
import jax.numpy as jnp
from jax.experimental import pallas as pl
from jax.experimental.pallas import tpu as pltpu

B, R, N, D = 64, 29, 5000, 512
TOP_K = 5
NEG_INF = float("-inf")


def _region_kernel(q_ref, db_ref, vals_ref, idx_ref):
    q = q_ref[0]
    db = db_ref[0]

    qn = jnp.sqrt(jnp.sum(q * q, axis=1, keepdims=True))
    qh = q / jnp.maximum(qn, 1e-12)

    dsq = db * db
    ssq = jnp.sum(dsq, axis=1, keepdims=True)
    dbh = db / jnp.maximum(jnp.sqrt(ssq), 1e-12)

    sims = jax.lax.dot_general(
        qh.astype(jnp.bfloat16), dbh.astype(jnp.bfloat16),
        (((1,), (1,)), ((), ())),
        preferred_element_type=jnp.float32,
    )

    lane = jax.lax.broadcasted_iota(jnp.int32, (B, N), 1)
    vals = []
    idxs = []
    s = sims
    for k in range(TOP_K):
        m = jnp.max(s, axis=1, keepdims=True)
        ix = jnp.min(jnp.where(s == m, lane, N), axis=1, keepdims=True)
        vals.append(m)
        idxs.append(ix)
        if k + 1 < TOP_K:
            s = jnp.where(lane == ix, NEG_INF, s)
    vals_ref[0] = jnp.concatenate(vals, axis=1)
    idx_ref[0] = jnp.concatenate(idxs, axis=1)


@functools.partial(jax.jit, static_argnames=())
def _run(qT, db):
    grid = (R,)
    vals_rbk, idx_rbk = pl.pallas_call(
        _region_kernel,
        grid=grid,
        in_specs=[
            pl.BlockSpec((1, B, D), lambda r: (r, 0, 0)),
            pl.BlockSpec((1, N, D), lambda r: (r, 0, 0)),
        ],
        out_specs=[
            pl.BlockSpec((1, B, TOP_K), lambda r: (r, 0, 0)),
            pl.BlockSpec((1, B, TOP_K), lambda r: (r, 0, 0)),
        ],
        out_shape=[
            jax.ShapeDtypeStruct((R, B, TOP_K), jnp.float32),
            jax.ShapeDtypeStruct((R, B, TOP_K), jnp.int32),
        ],
        compiler_params=pltpu.CompilerParams(
            dimension_semantics=("parallel",),
        ),
    )(qT, db)
    return vals_rbk, idx_rbk


def kernel(query_visual_features, region_features_db, top_k):
    qT = jnp.transpose(query_visual_features, (1, 0, 2))
    vals_rbk, idx_rbk = _run(qT, region_features_db)
    top_vals = jnp.transpose(vals_rbk, (1, 0, 2))
    top_idx = jnp.transpose(idx_rbk, (1, 0, 2))
    similarity_scores = top_vals[..., 0]
    return top_vals, top_idx, similarity_scores


# speedup vs baseline: 1.0468x; 1.0468x over previous
import functools

import jax
import jax.numpy as jnp
from jax.experimental import pallas as pl
from jax.experimental.pallas import tpu as pltpu

B, R, N, D = 64, 29, 5000, 512
TOP_K = 5
NEG_INF = float("-inf")


def _one_region(q, db):

    qn = jnp.sqrt(jnp.sum(q * q, axis=1, keepdims=True))
    qh = q / jnp.maximum(qn, 1e-12)

    dsq = db * db
    ssq = jnp.sum(dsq, axis=1, keepdims=True)
    dbh = db / jnp.maximum(jnp.sqrt(ssq), 1e-12)

    sims = jax.lax.dot_general(
        qh.astype(jnp.bfloat16), dbh.astype(jnp.bfloat16),
        (((1,), (1,)), ((), ())),
        preferred_element_type=jnp.float32,
    )

    lane = jax.lax.broadcasted_iota(jnp.int32, (B, N), 1)
    vals = []
    idxs = []
    s = sims
    for k in range(TOP_K):
        m = jnp.max(s, axis=1, keepdims=True)
        ix = jnp.argmax(s, axis=1, keepdims=True).astype(jnp.int32)
        vals.append(m)
        idxs.append(ix)
        if k + 1 < TOP_K:
            s = jnp.where(lane == ix, NEG_INF, s)
    return jnp.concatenate(vals, axis=1), jnp.concatenate(idxs, axis=1)


RPB = 2
GRID = R // RPB + 1


def _region_kernel(q_ref, db_ref, vals_ref, idx_ref):
    for j in range(RPB):
        v, ix = _one_region(q_ref[j], db_ref[j])
        vals_ref[j] = v
        idx_ref[j] = ix


@functools.partial(jax.jit, static_argnames=())
def _run(qT, db):
    vals_rbk, idx_rbk = pl.pallas_call(
        _region_kernel,
        grid=(GRID,),
        in_specs=[
            pl.BlockSpec((RPB, B, D), lambda r: (r, 0, 0)),
            pl.BlockSpec((RPB, N, D), lambda r: (r, 0, 0)),
        ],
        out_specs=[
            pl.BlockSpec((RPB, B, TOP_K), lambda r: (r, 0, 0)),
            pl.BlockSpec((RPB, B, TOP_K), lambda r: (r, 0, 0)),
        ],
        out_shape=[
            jax.ShapeDtypeStruct((GRID * RPB, B, TOP_K), jnp.float32),
            jax.ShapeDtypeStruct((GRID * RPB, B, TOP_K), jnp.int32),
        ],
        compiler_params=pltpu.CompilerParams(
            dimension_semantics=("parallel",),
            vmem_limit_bytes=110 * 1024 * 1024,
        ),
    )(qT, db)
    return vals_rbk[:R], idx_rbk[:R]


def kernel(query_visual_features, region_features_db, top_k):
    qT = jnp.transpose(query_visual_features, (1, 0, 2))
    vals_rbk, idx_rbk = _run(qT, region_features_db)
    top_vals = jnp.transpose(vals_rbk, (1, 0, 2))
    top_idx = jnp.transpose(idx_rbk, (1, 0, 2))
    similarity_scores = top_vals[..., 0]
    return top_vals, top_idx, similarity_scores


# speedup vs baseline: 1.0868x; 1.0382x over previous
import functools

import jax
import jax.numpy as jnp
from jax.experimental import pallas as pl
from jax.experimental.pallas import tpu as pltpu

B, R, N, D = 64, 29, 5000, 512
TOP_K = 5
NEG_INF = float("-inf")


def _region_kernel(q_ref, db_ref, vals_ref, idx_ref):
    q = q_ref[0]
    db = db_ref[0]

    qn = jnp.sqrt(jnp.sum(q * q, axis=1, keepdims=True))
    qh = q / jnp.maximum(qn, 1e-12)

    dsq = db * db
    ssq = jnp.sum(dsq, axis=1, keepdims=True)
    dbh = db / jnp.maximum(jnp.sqrt(ssq), 1e-12)

    sims = jax.lax.dot_general(
        qh.astype(jnp.bfloat16), dbh.astype(jnp.bfloat16),
        (((1,), (1,)), ((), ())),
        preferred_element_type=jnp.float32,
    )

    lane = jax.lax.broadcasted_iota(jnp.int32, (B, N), 1)
    vals = []
    idxs = []
    s = sims
    for k in range(TOP_K):
        m = jnp.max(s, axis=1, keepdims=True)
        ix = jnp.argmax(s, axis=1, keepdims=True).astype(jnp.int32)
        vals.append(m)
        idxs.append(ix)
        if k + 1 < TOP_K:
            s = jnp.where(lane == ix, NEG_INF, s)
    vals_ref[0] = jnp.concatenate(vals, axis=1)
    idx_ref[0] = jnp.concatenate(idxs, axis=1)


@functools.partial(jax.jit, static_argnames=())
def _run(qT, db):
    grid = (R,)
    vals_rbk, idx_rbk = pl.pallas_call(
        _region_kernel,
        grid=grid,
        in_specs=[
            pl.BlockSpec((1, B, D), lambda r: (r, 0, 0)),
            pl.BlockSpec((1, N, D), lambda r: (r, 0, 0)),
        ],
        out_specs=[
            pl.BlockSpec((1, B, TOP_K), lambda r: (r, 0, 0)),
            pl.BlockSpec((1, B, TOP_K), lambda r: (r, 0, 0)),
        ],
        out_shape=[
            jax.ShapeDtypeStruct((R, B, TOP_K), jnp.float32),
            jax.ShapeDtypeStruct((R, B, TOP_K), jnp.int32),
        ],
        compiler_params=pltpu.CompilerParams(
            dimension_semantics=("parallel",),
        ),
    )(qT, db)
    return vals_rbk, idx_rbk


def kernel(query_visual_features, region_features_db, top_k):
    qT = jnp.transpose(query_visual_features, (1, 0, 2))
    vals_rbk, idx_rbk = _run(qT, region_features_db)
    top_vals = jnp.transpose(vals_rbk, (1, 0, 2))
    top_idx = jnp.transpose(idx_rbk, (1, 0, 2))
    similarity_scores = top_vals[..., 0]
    return top_vals, top_idx, similarity_scores
